# baseline (speedup 1.0000x reference)
import functools

import jax
import jax.numpy as jnp
from jax import lax
from jax.experimental import pallas as pl
from jax.experimental.pallas import tpu as pltpu
from jax.experimental.pallas import tpu_sc as plsc

B, S = 16384, 200
N = B * S
NC, NS = 2, 16
NW = NC * NS
PER_W = N // NW
CHUNK = 25600
CHUNKS = PER_W // CHUNK

_mesh = plsc.VectorSubcoreMesh(core_axis_name="c", subcore_axis_name="s")


@functools.partial(
    pl.kernel,
    mesh=_mesh,
    out_type=jax.ShapeDtypeStruct((N,), jnp.float32),
    scratch_types=[
        pltpu.VMEM((CHUNK,), jnp.int32),
        pltpu.VMEM((CHUNK,), jnp.int32),
        pltpu.VMEM((CHUNK,), jnp.float32),
        pltpu.VMEM((CHUNK,), jnp.float32),
        pltpu.SemaphoreType.DMA,
        pltpu.SemaphoreType.DMA,
        pltpu.SemaphoreType.DMA,
        pltpu.SemaphoreType.DMA,
        pltpu.SemaphoreType.DMA,
    ],
)
def _gather_sc(idx_hbm, u_hbm, out_hbm, i0, i1, o0, o1, si0, si1, so0, so1, sg):
    wid = lax.axis_index("s") * NC + lax.axis_index("c")
    base0 = wid * PER_W
    idx_bufs, out_bufs = (i0, i1), (o0, o1)
    isems, osems = (si0, si1), (so0, so1)

    pltpu.async_copy(idx_hbm.at[pl.ds(base0, CHUNK)], i0, si0)
    for k in range(CHUNKS):
        cur = k & 1
        if k + 1 < CHUNKS:
            nxt = (k + 1) & 1
            pltpu.async_copy(
                idx_hbm.at[pl.ds(base0 + (k + 1) * CHUNK, CHUNK)],
                idx_bufs[nxt], isems[nxt])
        pltpu.make_async_copy(
            idx_hbm.at[pl.ds(base0 + k * CHUNK, CHUNK)],
            idx_bufs[cur], isems[cur]).wait()
        if k >= 2:
            pltpu.make_async_copy(
                out_bufs[cur],
                out_hbm.at[pl.ds(base0 + (k - 2) * CHUNK, CHUNK)],
                osems[cur]).wait()
        pltpu.async_copy(u_hbm.at[idx_bufs[cur]], out_bufs[cur], sg).wait()
        pltpu.async_copy(
            out_bufs[cur], out_hbm.at[pl.ds(base0 + k * CHUNK, CHUNK)],
            osems[cur])
    for k in (CHUNKS - 2, CHUNKS - 1):
        cur = k & 1
        pltpu.make_async_copy(
            out_bufs[cur], out_hbm.at[pl.ds(base0 + k * CHUNK, CHUNK)],
            osems[cur]).wait()


def kernel(idx, u):
    out = _gather_sc(idx.reshape(N), u)
    return out.reshape(idx.shape)


# speedup vs baseline: 1.5422x; 1.5422x over previous
import functools

import jax
import jax.numpy as jnp
from jax import lax
from jax.experimental import pallas as pl
from jax.experimental.pallas import tpu as pltpu
from jax.experimental.pallas import tpu_sc as plsc

B, S = 16384, 200
N = B * S
TAB = 1000000
NC, NS = 2, 16
NW = NC * NS
PER_W = N // NW
CHUNK = 25600
CHUNKS = PER_W // CHUNK
PIECE = 20000
PIECES = TAB // PIECE

_mesh = plsc.VectorSubcoreMesh(core_axis_name="c", subcore_axis_name="s")


@functools.partial(
    pl.kernel,
    mesh=_mesh,
    out_type=jax.ShapeDtypeStruct((N,), jnp.float32),
    scratch_types=[
        pltpu.VMEM_SHARED((TAB,), jnp.float32),
        pltpu.VMEM((CHUNK,), jnp.int32),
        pltpu.VMEM((CHUNK,), jnp.float32),
        pltpu.SemaphoreType.DMA,
    ],
)
def _gather_sc(idx_hbm, u_hbm, out_hbm, u_sp, idx_v, out_v, sem):
    sid = lax.axis_index("s")
    wid = sid * NC + lax.axis_index("c")
    base0 = wid * PER_W

    for j in range((PIECES + NS - 1) // NS):
        piece = sid + NS * j

        @pl.when(piece < PIECES)
        def _stage():
            off = piece * PIECE
            bounce = out_v.at[pl.ds(0, PIECE)]
            pltpu.sync_copy(u_hbm.at[pl.ds(off, PIECE)], bounce)
            pltpu.sync_copy(bounce, u_sp.at[pl.ds(off, PIECE)])

    plsc.subcore_barrier()

    for k in range(CHUNKS):
        base = base0 + k * CHUNK
        pltpu.sync_copy(idx_hbm.at[pl.ds(base, CHUNK)], idx_v)
        pltpu.async_copy(u_sp.at[idx_v], out_v, sem).wait()
        pltpu.sync_copy(out_v, out_hbm.at[pl.ds(base, CHUNK)])


def kernel(idx, u):
    out = _gather_sc(idx.reshape(N), u)
    return out.reshape(idx.shape)
